# baseline (speedup 1.0000x reference)
import jax
import jax.numpy as jnp
from jax.experimental import pallas as pl
from jax.experimental.pallas import tpu as pltpu

_N = 10000
_NPAD = 10240
_BM0 = 1024
_BK0 = 2048
_GI0 = _NPAD // _BM0
_GK0 = _NPAD // _BK0
_BM = 512
_GI = _NPAD // _BM

_BF = jnp.bfloat16


def _accum(k, acc_ref, part):
    @pl.when(k == 0)
    def _init():
        acc_ref[...] = part

    @pl.when(k > 0)
    def _acc():
        acc_ref[...] += part


def _pass0_body(x0_ref, a_ref, u_ref, cs_ref, a16_ref,
                x0bf_ref, acc_ref, csacc_ref):
    i = pl.program_id(0)
    k = pl.program_id(1)

    @pl.when((i == 0) & (k == 0))
    def _prologue():
        x0bf_ref[...] = x0_ref[...].astype(_BF)
        csacc_ref[...] = jnp.zeros_like(csacc_ref)

    boundary = (i == _GI0 - 1) | (k == _GK0 - 1)

    @pl.when(boundary)
    def _edge():
        a = a_ref[...]
        rows = jax.lax.broadcasted_iota(jnp.int32, (_BM0, _BK0), 0) + i * _BM0
        cols = jax.lax.broadcasted_iota(jnp.int32, (_BM0, _BK0), 1) + k * _BK0
        a = jnp.where((rows < _N) & (cols < _N), a, 0.0)
        _step0(a, k, a16_ref, csacc_ref, x0bf_ref, acc_ref)

    @pl.when(jnp.logical_not(boundary))
    def _interior():
        _step0(a_ref[...], k, a16_ref, csacc_ref, x0bf_ref, acc_ref)

    @pl.when(k == _GK0 - 1)
    def _epilogue():
        u_ref[...] = acc_ref[...]

    @pl.when((i == _GI0 - 1) & (k == _GK0 - 1))
    def _final():
        cs_ref[...] = csacc_ref[...]


def _step0(a, k, a16_ref, csacc_ref, x0bf_ref, acc_ref):
    a16 = a.astype(_BF)
    a16_ref[...] = a16
    csacc_ref[:, pl.ds(k * _BK0, _BK0)] += jnp.sum(a, axis=0, keepdims=True)
    part = jnp.dot(a16, x0bf_ref[pl.ds(k * _BK0, _BK0), :],
                   preferred_element_type=jnp.float32)
    _accum(k, acc_ref, part)


def _layer_tail(pool, w_ref, b_ref, dcol):
    lin = jnp.dot(pool.astype(_BF), w_ref[...].astype(_BF),
                  preferred_element_type=jnp.float32) + b_ref[...]
    return jnp.tanh(lin / dcol)


def _pass1_body(ab_ref, u0_ref, x0_ref, w0_ref, b0_ref, w_ref, b_ref,
                deg_ref, out_ref, xs_ref, xbf_ref):
    i = pl.program_id(0)

    @pl.when(i == 0)
    def _prologue():
        pool0 = u0_ref[...] + x0_ref[...]
        x1 = _layer_tail(pool0, w0_ref, b0_ref,
                         deg_ref[...].reshape(_NPAD, 1))
        xs_ref[...] = x1
        xbf_ref[...] = x1.astype(_BF)

    acc = jnp.dot(ab_ref[...], xbf_ref[...],
                  preferred_element_type=jnp.float32)
    pool = acc + xs_ref[pl.ds(i * _BM, _BM), :]
    dblk = deg_ref[:, pl.ds(i * _BM, _BM)].reshape(_BM, 1)
    out_ref[...] = _layer_tail(pool, w_ref, b_ref, dblk)


def _mid_body(ab_ref, xin_ref, w_ref, b_ref, deg_ref, out_ref, xbf_ref):
    i = pl.program_id(0)

    @pl.when(i == 0)
    def _prologue():
        xbf_ref[...] = xin_ref[...].astype(_BF)

    acc = jnp.dot(ab_ref[...], xbf_ref[...],
                  preferred_element_type=jnp.float32)
    pool = acc + xin_ref[pl.ds(i * _BM, _BM), :]
    dblk = deg_ref[:, pl.ds(i * _BM, _BM)].reshape(_BM, 1)
    out_ref[...] = _layer_tail(pool, w_ref, b_ref, dblk)


_PARAMS0 = pltpu.CompilerParams(
    dimension_semantics=("arbitrary", "arbitrary"))
_PARAMS1 = pltpu.CompilerParams(dimension_semantics=("arbitrary",))


def _pass0(x0p, adj):
    return pl.pallas_call(
        _pass0_body,
        grid=(_GI0, _GK0),
        in_specs=[
            pl.BlockSpec((_NPAD, 128), lambda i, k: (0, 0)),
            pl.BlockSpec((_BM0, _BK0), lambda i, k: (i, k)),
        ],
        out_specs=[
            pl.BlockSpec((_BM0, 128), lambda i, k: (i, 0)),
            pl.BlockSpec((1, _NPAD), lambda i, k: (0, 0)),
            pl.BlockSpec((_BM0, _BK0), lambda i, k: (i, k)),
        ],
        out_shape=[
            jax.ShapeDtypeStruct((_NPAD, 128), jnp.float32),
            jax.ShapeDtypeStruct((1, _NPAD), jnp.float32),
            jax.ShapeDtypeStruct((_NPAD, _NPAD), _BF),
        ],
        scratch_shapes=[
            pltpu.VMEM((_NPAD, 128), _BF),
            pltpu.VMEM((_BM0, 128), jnp.float32),
            pltpu.VMEM((1, _NPAD), jnp.float32),
        ],
        compiler_params=_PARAMS0,
    )(x0p, adj)


def _pass1(ab, u0, x0p, w0, b0, w, b, deg):
    return pl.pallas_call(
        _pass1_body,
        grid=(_GI,),
        in_specs=[
            pl.BlockSpec((_BM, _NPAD), lambda i: (i, 0)),
            pl.BlockSpec((_NPAD, 128), lambda i: (0, 0)),
            pl.BlockSpec((_NPAD, 128), lambda i: (0, 0)),
            pl.BlockSpec((128, 32), lambda i: (0, 0)),
            pl.BlockSpec((1, 32), lambda i: (0, 0)),
            pl.BlockSpec((32, 32), lambda i: (0, 0)),
            pl.BlockSpec((1, 32), lambda i: (0, 0)),
            pl.BlockSpec((1, _NPAD), lambda i: (0, 0)),
        ],
        out_specs=pl.BlockSpec((_BM, 32), lambda i: (i, 0)),
        out_shape=jax.ShapeDtypeStruct((_NPAD, 32), jnp.float32),
        scratch_shapes=[
            pltpu.VMEM((_NPAD, 32), jnp.float32),
            pltpu.VMEM((_NPAD, 32), _BF),
        ],
        compiler_params=_PARAMS1,
    )(ab, u0, x0p, w0, b0, w, b, deg)


def _mid(ab, xin, w, b, deg, dout):
    return pl.pallas_call(
        _mid_body,
        grid=(_GI,),
        in_specs=[
            pl.BlockSpec((_BM, _NPAD), lambda i: (i, 0)),
            pl.BlockSpec((_NPAD, 32), lambda i: (0, 0)),
            pl.BlockSpec((32, dout), lambda i: (0, 0)),
            pl.BlockSpec((1, dout), lambda i: (0, 0)),
            pl.BlockSpec((1, _NPAD), lambda i: (0, 0)),
        ],
        out_specs=pl.BlockSpec((_BM, dout), lambda i: (i, 0)),
        out_shape=jax.ShapeDtypeStruct((_NPAD, dout), jnp.float32),
        scratch_shapes=[
            pltpu.VMEM((_NPAD, 32), _BF),
        ],
        compiler_params=_PARAMS1,
    )(ab, xin, w, b, deg)


def kernel(node_feat, adjacency_matrix, batch_graph, W0, b0, W1, b1,
           W2, b2, W3, b3):
    del batch_graph
    x0p = jnp.pad(node_feat, ((0, _NPAD - _N), (0, 0)))
    u0, cs, ab = _pass0(x0p, adjacency_matrix)
    deg = cs + 1.0
    x2 = _pass1(ab, u0, x0p, W0, b0.reshape(1, 32), W1, b1.reshape(1, 32),
                deg)
    x3 = _mid(ab, x2, W2, b2.reshape(1, 32), deg, 32)
    x4 = _mid(ab, x3, W3, b3.reshape(1, 1), deg, 1)
    return x4[:_N, :]


# speedup vs baseline: 1.0305x; 1.0305x over previous
import jax
import jax.numpy as jnp
from jax.experimental import pallas as pl
from jax.experimental.pallas import tpu as pltpu

_N = 10000
_NPAD = 10240
_BM0 = 1024
_BK0 = 2048
_GI0 = _NPAD // _BM0
_GK0 = _NPAD // _BK0
_BM = 1024
_GI = _NPAD // _BM

_BF = jnp.bfloat16


def _accum(k, acc_ref, part):
    @pl.when(k == 0)
    def _init():
        acc_ref[...] = part

    @pl.when(k > 0)
    def _acc():
        acc_ref[...] += part


def _pass0_body(x0_ref, a_ref, u_ref, cs_ref, a16_ref,
                x0bf_ref, acc_ref, csacc_ref):
    i = pl.program_id(0)
    k = pl.program_id(1)

    @pl.when((i == 0) & (k == 0))
    def _prologue():
        x0bf_ref[...] = x0_ref[...].astype(_BF)
        csacc_ref[...] = jnp.zeros_like(csacc_ref)

    boundary = (i == _GI0 - 1) | (k == _GK0 - 1)

    @pl.when(boundary)
    def _edge():
        a = a_ref[...]
        rows = jax.lax.broadcasted_iota(jnp.int32, (_BM0, _BK0), 0) + i * _BM0
        cols = jax.lax.broadcasted_iota(jnp.int32, (_BM0, _BK0), 1) + k * _BK0
        a = jnp.where((rows < _N) & (cols < _N), a, 0.0)
        _step0(a, k, a16_ref, csacc_ref, x0bf_ref, acc_ref)

    @pl.when(jnp.logical_not(boundary))
    def _interior():
        _step0(a_ref[...], k, a16_ref, csacc_ref, x0bf_ref, acc_ref)

    @pl.when(k == _GK0 - 1)
    def _epilogue():
        u_ref[...] = acc_ref[...] + x0_ref[pl.ds(i * _BM0, _BM0), :]

    @pl.when((i == _GI0 - 1) & (k == _GK0 - 1))
    def _final():
        cs_ref[...] = csacc_ref[...]


def _step0(a, k, a16_ref, csacc_ref, x0bf_ref, acc_ref):
    a16 = a.astype(_BF)
    a16_ref[...] = a16
    csacc_ref[:, pl.ds(k * _BK0, _BK0)] += jnp.sum(a, axis=0, keepdims=True)
    part = jnp.dot(a16, x0bf_ref[pl.ds(k * _BK0, _BK0), :],
                   preferred_element_type=jnp.float32)
    _accum(k, acc_ref, part)


def _layer_tail(pool, w_ref, b_ref, dcol):
    lin = jnp.dot(pool.astype(_BF), w_ref[...].astype(_BF),
                  preferred_element_type=jnp.float32) + b_ref[...]
    return jnp.tanh(lin / dcol)


def _pass1_body(ab_ref, pool0_ref, w0_ref, b0_ref, w_ref, b_ref,
                deg_ref, out_ref, xs_ref, xbf_ref):
    i = pl.program_id(0)

    @pl.when(i == 0)
    def _prologue():
        x1 = _layer_tail(pool0_ref[...], w0_ref, b0_ref,
                         deg_ref[...].reshape(_NPAD, 1))
        xs_ref[...] = x1
        xbf_ref[...] = x1.astype(_BF)

    acc = jnp.dot(ab_ref[...], xbf_ref[...],
                  preferred_element_type=jnp.float32)
    pool = acc + xs_ref[pl.ds(i * _BM, _BM), :]
    dblk = deg_ref[:, pl.ds(i * _BM, _BM)].reshape(_BM, 1)
    out_ref[...] = _layer_tail(pool, w_ref, b_ref, dblk)


def _mid_body(ab_ref, xin_ref, w_ref, b_ref, deg_ref, out_ref, xbf_ref):
    i = pl.program_id(0)

    @pl.when(i == 0)
    def _prologue():
        xbf_ref[...] = xin_ref[...].astype(_BF)

    acc = jnp.dot(ab_ref[...], xbf_ref[...],
                  preferred_element_type=jnp.float32)
    pool = acc + xin_ref[pl.ds(i * _BM, _BM), :]
    dblk = deg_ref[:, pl.ds(i * _BM, _BM)].reshape(_BM, 1)
    out_ref[...] = _layer_tail(pool, w_ref, b_ref, dblk)


_PARAMS0 = pltpu.CompilerParams(
    dimension_semantics=("arbitrary", "arbitrary"))
_PARAMS1 = pltpu.CompilerParams(dimension_semantics=("arbitrary",))


def _pass0(x0p, adj):
    return pl.pallas_call(
        _pass0_body,
        grid=(_GI0, _GK0),
        in_specs=[
            pl.BlockSpec((_NPAD, 128), lambda i, k: (0, 0)),
            pl.BlockSpec((_BM0, _BK0), lambda i, k: (i, k)),
        ],
        out_specs=[
            pl.BlockSpec((_BM0, 128), lambda i, k: (i, 0)),
            pl.BlockSpec((1, _NPAD), lambda i, k: (0, 0)),
            pl.BlockSpec((_BM0, _BK0), lambda i, k: (i, k)),
        ],
        out_shape=[
            jax.ShapeDtypeStruct((_NPAD, 128), jnp.float32),
            jax.ShapeDtypeStruct((1, _NPAD), jnp.float32),
            jax.ShapeDtypeStruct((_NPAD, _NPAD), _BF),
        ],
        scratch_shapes=[
            pltpu.VMEM((_NPAD, 128), _BF),
            pltpu.VMEM((_BM0, 128), jnp.float32),
            pltpu.VMEM((1, _NPAD), jnp.float32),
        ],
        compiler_params=_PARAMS0,
    )(x0p, adj)


def _pass1(ab, pool0, w0, b0, w, b, deg):
    return pl.pallas_call(
        _pass1_body,
        grid=(_GI,),
        in_specs=[
            pl.BlockSpec((_BM, _NPAD), lambda i: (i, 0)),
            pl.BlockSpec((_NPAD, 128), lambda i: (0, 0)),
            pl.BlockSpec((128, 32), lambda i: (0, 0)),
            pl.BlockSpec((1, 32), lambda i: (0, 0)),
            pl.BlockSpec((32, 32), lambda i: (0, 0)),
            pl.BlockSpec((1, 32), lambda i: (0, 0)),
            pl.BlockSpec((1, _NPAD), lambda i: (0, 0)),
        ],
        out_specs=pl.BlockSpec((_BM, 32), lambda i: (i, 0)),
        out_shape=jax.ShapeDtypeStruct((_NPAD, 32), jnp.float32),
        scratch_shapes=[
            pltpu.VMEM((_NPAD, 32), jnp.float32),
            pltpu.VMEM((_NPAD, 32), _BF),
        ],
        compiler_params=_PARAMS1,
    )(ab, pool0, w0, b0, w, b, deg)


def _mid(ab, xin, w, b, deg, dout):
    return pl.pallas_call(
        _mid_body,
        grid=(_GI,),
        in_specs=[
            pl.BlockSpec((_BM, _NPAD), lambda i: (i, 0)),
            pl.BlockSpec((_NPAD, 32), lambda i: (0, 0)),
            pl.BlockSpec((32, dout), lambda i: (0, 0)),
            pl.BlockSpec((1, dout), lambda i: (0, 0)),
            pl.BlockSpec((1, _NPAD), lambda i: (0, 0)),
        ],
        out_specs=pl.BlockSpec((_BM, dout), lambda i: (i, 0)),
        out_shape=jax.ShapeDtypeStruct((_NPAD, dout), jnp.float32),
        scratch_shapes=[
            pltpu.VMEM((_NPAD, 32), _BF),
        ],
        compiler_params=_PARAMS1,
    )(ab, xin, w, b, deg)


def kernel(node_feat, adjacency_matrix, batch_graph, W0, b0, W1, b1,
           W2, b2, W3, b3):
    del batch_graph
    x0p = jnp.pad(node_feat, ((0, _NPAD - _N), (0, 0)))
    pool0, cs, ab = _pass0(x0p, adjacency_matrix)
    deg = cs + 1.0
    x2 = _pass1(ab, pool0, W0, b0.reshape(1, 32), W1, b1.reshape(1, 32),
                deg)
    x3 = _mid(ab, x2, W2, b2.reshape(1, 32), deg, 32)
    x4 = _mid(ab, x3, W3, b3.reshape(1, 1), deg, 1)
    return x4[:_N, :]
